# baseline (speedup 1.0000x reference)
import functools

import jax
import jax.numpy as jnp
from jax import lax
from jax.experimental import pallas as pl
from jax.experimental.pallas import tpu as pltpu
from jax.experimental.pallas import tpu_sc as plsc

N = 100000
C = 9
V = 200
D = 512
W = D // 2
B = 8
G = B * C
NBLK = N // B
NW = 32


@functools.partial(
    pl.kernel,
    out_type=jax.ShapeDtypeStruct((N, W), jnp.int32),
    mesh=plsc.VectorSubcoreMesh(core_axis_name="c", subcore_axis_name="s"),
    compiler_params=pltpu.CompilerParams(use_tc_tiling_on_sc=True),
    scratch_types=[
        pltpu.VMEM((2 * G,), jnp.int32),
        pltpu.VMEM((2, 2 * G, W), jnp.bfloat16),
        pltpu.VMEM((2 * B, W), jnp.bfloat16),
        pltpu.SemaphoreType.DMA((2,)),
        pltpu.SemaphoreType.DMA((2,)),
    ],
)
def _atom_encoder_sc(idx_hbm, tabs_hbm, out_hbm, idx_v, rows_v, out_v,
                     sem_idx, sem_g):
    w = lax.axis_index("s") * 2 + lax.axis_index("c")
    nblk_w = jnp.where(w < 20, 391, 390)
    blk0 = w * 390 + jnp.minimum(w, 20)

    def idx_copy(blk, slot, sem):
        return pltpu.make_async_copy(
            idx_hbm.at[pl.ds(blk * G, G)],
            idx_v.at[pl.ds(pl.multiple_of(slot * G, 8), G)], sem)

    def gather(slot, sem):
        return pltpu.make_async_copy(
            tabs_hbm.at[idx_v.at[pl.ds(pl.multiple_of(slot * G, 8), G)]],
            rows_v.at[slot].bitcast(jnp.int32), sem)

    idx_copy(blk0, 0, sem_idx.at[0]).start()
    idx_copy(blk0, 0, sem_idx.at[0]).wait()
    gather(0, sem_g.at[0]).start()

    @pl.when(nblk_w > 1)
    def _():
        idx_copy(blk0 + 1, 1, sem_idx.at[1]).start()

    def block_step(k, carry):
        blk = blk0 + k
        buf = lax.rem(k, 2)
        nbuf = 1 - buf

        gather(buf, sem_g.at[buf]).wait()

        @pl.when(k + 2 < nblk_w)
        def _():
            idx_copy(blk + 2, buf, sem_idx.at[buf]).start()

        @pl.when(k + 1 < nblk_w)
        def _():
            idx_copy(blk + 1, nbuf, sem_idx.at[nbuf]).wait()
            gather(nbuf, sem_g.at[nbuf]).start()

        def row_step(n, c1):
            def col_step(c, c2):
                sl = pl.ds(c * 16, 16)
                r0 = pl.multiple_of(n * 2 * C, 2)
                acc = rows_v[buf, pl.ds(r0, 2), sl]
                for i in range(1, C):
                    acc = acc + rows_v[buf, pl.ds(r0 + 2 * i, 2), sl]
                out_v[pl.ds(pl.multiple_of(n * 2, 2), 2), sl] = acc
                return c2

            return lax.fori_loop(0, W // 16, col_step, c1)

        lax.fori_loop(0, B, row_step, 0)
        pltpu.sync_copy(out_v.bitcast(jnp.int32),
                        out_hbm.at[pl.ds(blk * B, B)])
        return carry

    lax.fori_loop(0, nblk_w, block_step, 0)


def kernel(x, tables):
    offs = (jnp.arange(C, dtype=jnp.int32) * V)[None, :]
    idx = (x.astype(jnp.int32) + offs).reshape(N * C)
    tabs16 = tables.astype(jnp.bfloat16).reshape(C * V, W, 2)
    tabs = lax.bitcast_convert_type(tabs16, jnp.int32)
    packed = _atom_encoder_sc(idx, tabs)
    out16 = lax.bitcast_convert_type(packed, jnp.bfloat16)
    return out16.reshape(N, D).astype(jnp.float32)


# speedup vs baseline: 1.9690x; 1.9690x over previous
import functools

import jax
import jax.numpy as jnp
from jax import lax
from jax.experimental import pallas as pl
from jax.experimental.pallas import tpu as pltpu
from jax.experimental.pallas import tpu_sc as plsc

N = 100000
C = 9
V = 200
D = 512
W = D // 2
B = 8
G = B * C
NBLK = N // B
NW = 32
L = 16

HI = -65536


@functools.partial(
    pl.kernel,
    out_type=jax.ShapeDtypeStruct((N, D), jnp.float32),
    mesh=plsc.VectorSubcoreMesh(core_axis_name="c", subcore_axis_name="s"),
    scratch_types=[
        pltpu.VMEM((2 * G,), jnp.int32),
        pltpu.VMEM((2, G, W), jnp.int32),
        pltpu.VMEM((B, D), jnp.float32),
        pltpu.SemaphoreType.DMA((2,)),
        pltpu.SemaphoreType.DMA((2,)),
    ],
)
def _atom_encoder_sc(idx_hbm, tabs_hbm, out_hbm, idx_v, rows_v, out_v,
                     sem_idx, sem_g):
    w = lax.axis_index("s") * 2 + lax.axis_index("c")
    nblk_w = jnp.where(w < 20, 391, 390)
    blk0 = w * 390 + jnp.minimum(w, 20)

    def idx_copy(blk, slot, sem):
        return pltpu.make_async_copy(
            idx_hbm.at[pl.ds(blk * G, G)],
            idx_v.at[pl.ds(pl.multiple_of(slot * G, 8), G)], sem)

    def gather(slot, sem):
        return pltpu.make_async_copy(
            tabs_hbm.at[idx_v.at[pl.ds(pl.multiple_of(slot * G, 8), G)]],
            rows_v.at[slot], sem)

    idx_copy(blk0, 0, sem_idx.at[0]).start()
    idx_copy(blk0, 0, sem_idx.at[0]).wait()
    gather(0, sem_g.at[0]).start()

    @pl.when(nblk_w > 1)
    def _():
        idx_copy(blk0 + 1, 1, sem_idx.at[1]).start()

    def block_step(k, carry):
        blk = blk0 + k
        buf = lax.rem(k, 2)
        nbuf = 1 - buf

        gather(buf, sem_g.at[buf]).wait()

        @pl.when(k + 2 < nblk_w)
        def _():
            idx_copy(blk + 2, buf, sem_idx.at[buf]).start()

        @pl.when(k + 1 < nblk_w)
        def _():
            idx_copy(blk + 1, nbuf, sem_idx.at[nbuf]).wait()
            gather(nbuf, sem_g.at[nbuf]).start()

        def row_step(n, c1):
            def col_step(c, c2):
                sl = pl.ds(c * L, L)
                wd = rows_v[buf, n * C, sl]
                lo = lax.bitcast_convert_type(wd << 16, jnp.float32)
                hi = lax.bitcast_convert_type(wd & HI, jnp.float32)
                for i in range(1, C):
                    wd = rows_v[buf, n * C + i, sl]
                    lo = lo + lax.bitcast_convert_type(wd << 16, jnp.float32)
                    hi = hi + lax.bitcast_convert_type(wd & HI, jnp.float32)
                out_v[n, sl] = lo
                out_v[n, pl.ds(W + c * L, L)] = hi
                return c2

            return lax.fori_loop(0, W // L, col_step, c1)

        lax.fori_loop(0, B, row_step, 0)
        pltpu.sync_copy(out_v, out_hbm.at[pl.ds(blk * B, B)])
        return carry

    lax.fori_loop(0, nblk_w, block_step, 0)


def kernel(x, tables):
    offs = (jnp.arange(C, dtype=jnp.int32) * V)[None, :]
    idx = (x.astype(jnp.int32) + offs).reshape(N * C)
    tabs16 = tables.astype(jnp.bfloat16).reshape(C * V, D)
    pairs = jnp.stack([tabs16[:, :W], tabs16[:, W:]], axis=-1)
    tabs = lax.bitcast_convert_type(pairs, jnp.int32)
    return _atom_encoder_sc(idx, tabs)


# speedup vs baseline: 2.1651x; 1.0996x over previous
import functools

import jax
import jax.numpy as jnp
from jax import lax
from jax.experimental import pallas as pl
from jax.experimental.pallas import tpu as pltpu
from jax.experimental.pallas import tpu_sc as plsc

N = 100000
C = 9
V = 200
D = 512
W = D // 2
B = 8
G = B * C
NBLK = N // B
NW = 32
L = 16

@functools.partial(
    pl.kernel,
    out_type=jax.ShapeDtypeStruct((N, D), jnp.float32),
    mesh=plsc.VectorSubcoreMesh(core_axis_name="c", subcore_axis_name="s"),
    scratch_types=[
        pltpu.VMEM((2 * G,), jnp.int32),
        pltpu.VMEM((2, G, W), jnp.int32),
        pltpu.VMEM((2, B, D), jnp.float32),
        pltpu.SemaphoreType.DMA((2,)),
        pltpu.SemaphoreType.DMA((2,)),
        pltpu.SemaphoreType.DMA((2,)),
    ],
)
def _atom_encoder_sc(idx_hbm, tabs_hbm, out_hbm, idx_v, rows_v, out_v,
                     sem_idx, sem_g, sem_o):
    w = lax.axis_index("s") * 2 + lax.axis_index("c")
    nblk_w = jnp.where(w < 20, 391, 390)
    blk0 = w * 390 + jnp.minimum(w, 20)

    def idx_copy(blk, slot, sem):
        return pltpu.make_async_copy(
            idx_hbm.at[pl.ds(blk * G, G)],
            idx_v.at[pl.ds(pl.multiple_of(slot * G, 8), G)], sem)

    def gather(slot, sem):
        return pltpu.make_async_copy(
            tabs_hbm.at[idx_v.at[pl.ds(pl.multiple_of(slot * G, 8), G)]],
            rows_v.at[slot], sem)

    def out_store(slot, blk, sem):
        return pltpu.make_async_copy(
            out_v.at[slot], out_hbm.at[pl.ds(blk * B, B)], sem)

    idx_copy(blk0, 0, sem_idx.at[0]).start()
    idx_copy(blk0, 0, sem_idx.at[0]).wait()
    gather(0, sem_g.at[0]).start()

    @pl.when(nblk_w > 1)
    def _():
        idx_copy(blk0 + 1, 1, sem_idx.at[1]).start()

    def block_step(k, carry):
        blk = blk0 + k
        buf = lax.rem(k, 2)
        nbuf = 1 - buf

        gather(buf, sem_g.at[buf]).wait()

        @pl.when(k + 2 < nblk_w)
        def _():
            idx_copy(blk + 2, buf, sem_idx.at[buf]).start()

        @pl.when(k + 1 < nblk_w)
        def _():
            idx_copy(blk + 1, nbuf, sem_idx.at[nbuf]).wait()
            gather(nbuf, sem_g.at[nbuf]).start()

        @pl.when(k >= 2)
        def _():
            out_store(buf, blk - 2, sem_o.at[buf]).wait()

        def row_step(n, c1):
            def col_step(c, c2):
                for u in range(2):
                    cc = c * 2 + u
                    sl = pl.ds(cc * L, L)
                    wd = rows_v[buf, n * C, sl]
                    lo = lax.bitcast_convert_type(wd << 16, jnp.float32)
                    hi = lax.bitcast_convert_type(wd, jnp.float32)
                    for i in range(1, C):
                        wd = rows_v[buf, n * C + i, sl]
                        lo = lo + lax.bitcast_convert_type(
                            wd << 16, jnp.float32)
                        hi = hi + lax.bitcast_convert_type(wd, jnp.float32)
                    out_v[buf, n, sl] = lo
                    out_v[buf, n, pl.ds(W + cc * L, L)] = hi
                return c2

            return lax.fori_loop(0, W // L // 2, col_step, c1)

        lax.fori_loop(0, B, row_step, 0)
        out_store(buf, blk, sem_o.at[buf]).start()
        return carry

    lax.fori_loop(0, nblk_w, block_step, 0)

    p = lax.rem(nblk_w, 2)
    out_store(p, blk0 + nblk_w - 2, sem_o.at[p]).wait()
    out_store(1 - p, blk0 + nblk_w - 1, sem_o.at[1 - p]).wait()


def kernel(x, tables):
    offs = (jnp.arange(C, dtype=jnp.int32) * V)[None, :]
    idx = (x.astype(jnp.int32) + offs).reshape(N * C)
    tabs16 = tables.astype(jnp.bfloat16).reshape(C * V, D)
    pairs = jnp.stack([tabs16[:, :W], tabs16[:, W:]], axis=-1)
    tabs = lax.bitcast_convert_type(pairs, jnp.int32)
    return _atom_encoder_sc(idx, tabs)


# speedup vs baseline: 2.1706x; 1.0026x over previous
import functools

import jax
import jax.numpy as jnp
from jax import lax
from jax.experimental import pallas as pl
from jax.experimental.pallas import tpu as pltpu
from jax.experimental.pallas import tpu_sc as plsc

N = 100000
C = 9
V = 200
D = 512
W = D // 2
B = 16
G = B * C
H = G // 2
NBLK = N // B
NW = 32
L = 16

@functools.partial(
    pl.kernel,
    out_type=jax.ShapeDtypeStruct((N, D), jnp.float32),
    mesh=plsc.VectorSubcoreMesh(core_axis_name="c", subcore_axis_name="s"),
    scratch_types=[
        pltpu.VMEM((2 * G,), jnp.int32),
        pltpu.VMEM((2, G, W), jnp.int32),
        pltpu.VMEM((2, B, D), jnp.float32),
        pltpu.SemaphoreType.DMA((2,)),
        pltpu.SemaphoreType.DMA((2,)),
        pltpu.SemaphoreType.DMA((2,)),
    ],
)
def _atom_encoder_sc(idx_hbm, tabs_hbm, out_hbm, idx_v, rows_v, out_v,
                     sem_idx, sem_g, sem_o):
    w = lax.axis_index("s") * 2 + lax.axis_index("c")
    nblk_w = jnp.where(w < 10, 196, 195)
    blk0 = w * 195 + jnp.minimum(w, 10)

    def idx_copy(blk, slot, sem):
        return pltpu.make_async_copy(
            idx_hbm.at[pl.ds(blk * G, G)],
            idx_v.at[pl.ds(pl.multiple_of(slot * G, 8), G)], sem)

    def gathers(slot, sem):
        return [
            pltpu.make_async_copy(
                tabs_hbm.at[idx_v.at[
                    pl.ds(pl.multiple_of(slot * G + h * H, 8), H)]],
                rows_v.at[slot, pl.ds(h * H, H)], sem)
            for h in range(2)
        ]

    def out_store(slot, blk, sem):
        return pltpu.make_async_copy(
            out_v.at[slot], out_hbm.at[pl.ds(blk * B, B)], sem)

    idx_copy(blk0, 0, sem_idx.at[0]).start()
    idx_copy(blk0, 0, sem_idx.at[0]).wait()
    for g in gathers(0, sem_g.at[0]):
        g.start()

    @pl.when(nblk_w > 1)
    def _():
        idx_copy(blk0 + 1, 1, sem_idx.at[1]).start()

    def block_step(k, carry):
        blk = blk0 + k
        buf = lax.rem(k, 2)
        nbuf = 1 - buf

        for g in gathers(buf, sem_g.at[buf]):
            g.wait()

        @pl.when(k + 2 < nblk_w)
        def _():
            idx_copy(blk + 2, buf, sem_idx.at[buf]).start()

        @pl.when(k + 1 < nblk_w)
        def _():
            idx_copy(blk + 1, nbuf, sem_idx.at[nbuf]).wait()
            for g in gathers(nbuf, sem_g.at[nbuf]):
                g.start()

        @pl.when(k >= 2)
        def _():
            out_store(buf, blk - 2, sem_o.at[buf]).wait()

        def row_step(n, c1):
            def col_step(c, c2):
                for u in range(2):
                    cc = c * 2 + u
                    sl = pl.ds(cc * L, L)
                    wd = rows_v[buf, n * C, sl]
                    lo = lax.bitcast_convert_type(wd << 16, jnp.float32)
                    hi = lax.bitcast_convert_type(wd, jnp.float32)
                    for i in range(1, C):
                        wd = rows_v[buf, n * C + i, sl]
                        lo = lo + lax.bitcast_convert_type(
                            wd << 16, jnp.float32)
                        hi = hi + lax.bitcast_convert_type(wd, jnp.float32)
                    out_v[buf, n, sl] = lo
                    out_v[buf, n, pl.ds(W + cc * L, L)] = hi
                return c2

            return lax.fori_loop(0, W // L // 2, col_step, c1)

        lax.fori_loop(0, B, row_step, 0)
        out_store(buf, blk, sem_o.at[buf]).start()
        return carry

    lax.fori_loop(0, nblk_w, block_step, 0)

    p = lax.rem(nblk_w, 2)
    out_store(p, blk0 + nblk_w - 2, sem_o.at[p]).wait()
    out_store(1 - p, blk0 + nblk_w - 1, sem_o.at[1 - p]).wait()


def kernel(x, tables):
    offs = (jnp.arange(C, dtype=jnp.int32) * V)[None, :]
    idx = (x.astype(jnp.int32) + offs).reshape(N * C)
    tabs16 = tables.astype(jnp.bfloat16).reshape(C * V, D)
    pairs = jnp.stack([tabs16[:, :W], tabs16[:, W:]], axis=-1)
    tabs = lax.bitcast_convert_type(pairs, jnp.int32)
    return _atom_encoder_sc(idx, tabs)
